# baseline (speedup 1.0000x reference)
import functools

import jax
import jax.numpy as jnp
from jax import lax
from jax.experimental import pallas as pl
from jax.experimental.pallas import tpu as pltpu
from jax.experimental.pallas import tpu_sc as plsc

VOCAB = 1000
SEQ = 50
BATCH = 1024
NUM_CORES = 2
NUM_SUBCORES = 16
NUM_WORKERS = NUM_CORES * NUM_SUBCORES
LANES = 16

VCHUNK = 40
CHUNKS_PER_SLAB = VOCAB // VCHUNK
N_UNITS = SEQ * CHUNKS_PER_SLAB
N_REGIONS = 3
N_COLS = 3


def _onehot_body(xt_hbm, t_hbm, xcols, buf, sem0, sem1, sem2):
    wid = lax.axis_index("s") * NUM_CORES + lax.axis_index("c")

    start = wid * N_UNITS // NUM_WORKERS
    n = (wid + 1) * N_UNITS // NUM_WORKERS - start
    l_lo = jnp.minimum(start // CHUNKS_PER_SLAB, SEQ - N_COLS)

    for j in range(N_COLS):
        pltpu.async_copy(
            xt_hbm.at[l_lo + j], xcols.at[pl.ds(j * BATCH, BATCH)], sem1
        )

    zeros16 = jnp.zeros((LANES,), jnp.float32)
    ones16 = jnp.full((LANES,), 1.0, jnp.float32)
    lane = lax.iota(jnp.int32, LANES)

    def zero_region(base_row):
        def zero_row(r, carry):
            for j in range(BATCH // LANES):
                buf[r + base_row, pl.ds(j * LANES, LANES)] = zeros16
            return carry

        lax.fori_loop(0, VCHUNK, zero_row, 0)

    def unit_params(u):
        l = u // CHUNKS_PER_SLAB
        c = u - l * CHUNKS_PER_SLAB
        v0 = pl.multiple_of(jnp.minimum(c * VCHUNK, VOCAB - VCHUNK), 8)
        return l, v0

    def scan_pass(li, v0, base_row, val16):
        def group8(i, carry):
            for gg in range(8):
                g = i * 8 + gg
                xs = xcols[pl.ds(li * BATCH + g * LANES, LANES)]
                rel = xs - v0
                mask = (rel >= 0) & (rel < VCHUNK)
                plsc.store_scatter(
                    buf, [rel + base_row, lane + g * LANES], val16, mask=mask
                )
            return carry

        lax.fori_loop(0, BATCH // LANES // 8, group8, 0)

    def do_unit(k, base_row, sem, first=False):
        u = start + k
        l, v0 = unit_params(u)

        if not first:
            pltpu.make_async_copy(
                buf.at[pl.ds(base_row, VCHUNK)],
                t_hbm.at[0, pl.ds(0, VCHUNK)],
                sem,
            ).wait()
            lp, v0p = unit_params(u - N_REGIONS)
            scan_pass(lp - l_lo, v0p, base_row, zeros16)

        scan_pass(l - l_lo, v0, base_row, ones16)
        pltpu.async_copy(
            buf.at[pl.ds(base_row, VCHUNK)],
            t_hbm.at[l, pl.ds(v0, VCHUNK)],
            sem,
        )

    sems = (sem0, sem1, sem2)

    zero_region(0)
    for j in range(N_COLS):
        pltpu.make_async_copy(
            xt_hbm.at[l_lo + j], xcols.at[pl.ds(j * BATCH, BATCH)], sem1
        ).wait()
    do_unit(0, 0, sem0, first=True)
    zero_region(VCHUNK)
    do_unit(1, VCHUNK, sem1, first=True)
    zero_region(2 * VCHUNK)
    do_unit(2, 2 * VCHUNK, sem2, first=True)

    def ring_body(p, carry):
        for r in range(N_REGIONS):
            do_unit(N_REGIONS + p * N_REGIONS + r, r * VCHUNK, sems[r])
        return carry

    lax.fori_loop(0, (n - N_REGIONS) // N_REGIONS, ring_body, 0)

    rem = (n - N_REGIONS) % N_REGIONS

    @pl.when(rem >= 1)
    def _():
        do_unit(n - rem, 0, sem0)

    @pl.when(rem == 2)
    def _():
        do_unit(n - 1, VCHUNK, sem1)

    for r in range(N_REGIONS):
        pltpu.make_async_copy(
            buf.at[pl.ds(r * VCHUNK, VCHUNK)],
            t_hbm.at[0, pl.ds(0, VCHUNK)],
            sems[r],
        ).wait()


def kernel(x):
    B, L = x.shape

    run = functools.partial(
        pl.kernel,
        mesh=plsc.VectorSubcoreMesh(core_axis_name="c", subcore_axis_name="s"),
        out_type=jax.ShapeDtypeStruct((L, VOCAB, B), jnp.float32),
        scratch_types=[
            pltpu.VMEM((N_COLS * B,), jnp.int32),
            pltpu.VMEM((N_REGIONS * VCHUNK, B), jnp.float32),
            pltpu.SemaphoreType.DMA,
            pltpu.SemaphoreType.DMA,
            pltpu.SemaphoreType.DMA,
        ],
        compiler_params=pltpu.CompilerParams(needs_layout_passes=False),
    )(_onehot_body)

    t = run(x.T)
    return jnp.transpose(t, (2, 0, 1))


# speedup vs baseline: 1.0146x; 1.0146x over previous
import functools

import jax
import jax.numpy as jnp
from jax import lax
from jax.experimental import pallas as pl
from jax.experimental.pallas import tpu as pltpu
from jax.experimental.pallas import tpu_sc as plsc

VOCAB = 1000
SEQ = 50
BATCH = 1024
NUM_CORES = 2
NUM_SUBCORES = 16
NUM_WORKERS = NUM_CORES * NUM_SUBCORES
LANES = 16

VCHUNK = 56
CHUNKS_PER_SLAB = -(-VOCAB // VCHUNK)
N_UNITS = SEQ * CHUNKS_PER_SLAB
N_COLS = 3


def _onehot_body(xt_hbm, t_hbm, xcols, buf, sem0, sem1):
    wid = lax.axis_index("s") * NUM_CORES + lax.axis_index("c")

    start = wid * N_UNITS // NUM_WORKERS
    n = (wid + 1) * N_UNITS // NUM_WORKERS - start
    l_lo = jnp.minimum(start // CHUNKS_PER_SLAB, SEQ - N_COLS)

    for j in range(N_COLS):
        pltpu.async_copy(
            xt_hbm.at[l_lo + j], xcols.at[pl.ds(j * BATCH, BATCH)], sem1
        )

    zeros16 = jnp.zeros((LANES,), jnp.float32)
    ones16 = jnp.full((LANES,), 1.0, jnp.float32)
    lane = lax.iota(jnp.int32, LANES)

    def zero_region(base_row):
        def zero_row(r, carry):
            for j in range(BATCH // LANES):
                buf[r + base_row, pl.ds(j * LANES, LANES)] = zeros16
            return carry

        lax.fori_loop(0, VCHUNK, zero_row, 0)

    def unit_params(u):
        l = u // CHUNKS_PER_SLAB
        c = u - l * CHUNKS_PER_SLAB
        v0 = pl.multiple_of(jnp.minimum(c * VCHUNK, VOCAB - VCHUNK), 8)
        return l, v0

    def scan_pass(li, v0, base_row, val16):
        def group8(i, carry):
            for gg in range(8):
                g = i * 8 + gg
                xs = xcols[pl.ds(li * BATCH + g * LANES, LANES)]
                rel = xs - v0
                mask = (rel >= 0) & (rel < VCHUNK)
                plsc.store_scatter(
                    buf, [rel + base_row, lane + g * LANES], val16, mask=mask
                )
            return carry

        lax.fori_loop(0, BATCH // LANES // 8, group8, 0)

    def do_unit(k, base_row, sem, first=False):
        u = start + k
        l, v0 = unit_params(u)

        if not first:
            pltpu.make_async_copy(
                buf.at[pl.ds(base_row, VCHUNK)],
                t_hbm.at[0, pl.ds(0, VCHUNK)],
                sem,
            ).wait()
            lp, v0p = unit_params(u - 2)
            scan_pass(lp - l_lo, v0p, base_row, zeros16)

        scan_pass(l - l_lo, v0, base_row, ones16)
        pltpu.async_copy(
            buf.at[pl.ds(base_row, VCHUNK)],
            t_hbm.at[l, pl.ds(v0, VCHUNK)],
            sem,
        )

    zero_region(0)
    for j in range(N_COLS):
        pltpu.make_async_copy(
            xt_hbm.at[l_lo + j], xcols.at[pl.ds(j * BATCH, BATCH)], sem1
        ).wait()
    do_unit(0, 0, sem0, first=True)
    zero_region(VCHUNK)
    do_unit(1, VCHUNK, sem1, first=True)

    def pair_body(p, carry):
        do_unit(2 + p * 2, 0, sem0)
        do_unit(3 + p * 2, VCHUNK, sem1)
        return carry

    lax.fori_loop(0, (n - 2) // 2, pair_body, 0)

    @pl.when(n % 2 == 1)
    def _():
        do_unit(n - 1, 0, sem0)

    pltpu.make_async_copy(
        buf.at[pl.ds(0, VCHUNK)], t_hbm.at[0, pl.ds(0, VCHUNK)], sem0
    ).wait()
    pltpu.make_async_copy(
        buf.at[pl.ds(VCHUNK, VCHUNK)], t_hbm.at[0, pl.ds(0, VCHUNK)], sem1
    ).wait()


def kernel(x):
    B, L = x.shape

    run = functools.partial(
        pl.kernel,
        mesh=plsc.VectorSubcoreMesh(core_axis_name="c", subcore_axis_name="s"),
        out_type=jax.ShapeDtypeStruct((L, VOCAB, B), jnp.float32),
        scratch_types=[
            pltpu.VMEM((N_COLS * B,), jnp.int32),
            pltpu.VMEM((2 * VCHUNK, B), jnp.float32),
            pltpu.SemaphoreType.DMA,
            pltpu.SemaphoreType.DMA,
        ],
        compiler_params=pltpu.CompilerParams(needs_layout_passes=False),
    )(_onehot_body)

    t = run(x.T)
    return jnp.transpose(t, (2, 0, 1))
